# Initial kernel scaffold; baseline (speedup 1.0000x reference)
import jax
import jax.numpy as jnp
from jax.experimental import pallas as pl


def kernel(x):
    raise NotImplementedError("write your pallas kernel here")


# speedup vs baseline: 9.7215x; 9.7215x over previous
import functools

import jax
import jax.numpy as jnp
from jax import lax
from jax.experimental import pallas as pl
from jax.experimental.pallas import tpu as pltpu
from jax.experimental.pallas import tpu_sc as plsc

B, F = 128, 32768
N = B * F
TOTAL_K = 8192
NW = 32
PER_W = N // NW
CH = 16384
NBINS = 4096
HIST = 16 * NBINS
LANES = 16

_mesh = plsc.VectorSubcoreMesh(
    core_axis_name="c", subcore_axis_name="s", num_cores=2, num_subcores=16
)


def _key_i32(u):
    return u ^ ((u >> 31) & jnp.int32(0x7FFFFFFF))


def _make_sc_hist(shift, bias, mshift, masked):
    def body(*refs):
        if masked:
            x_hbm, mt_hbm, out_hbm, hist_v, buf_v, mt_v, sem = refs
        else:
            x_hbm, out_hbm, hist_v, buf_v, mt_v, sem = refs
        wid = lax.axis_index("c") * 16 + lax.axis_index("s")
        base = wid * PER_W

        zeros16 = jnp.zeros((LANES,), jnp.int32)

        @pl.loop(0, HIST, step=LANES)
        def _zero(i):
            hist_v[pl.ds(i, LANES)] = zeros16

        if masked:
            pltpu.sync_copy(mt_hbm, mt_v)
        mt = mt_v[...] if masked else None

        lane_base = lax.iota(jnp.int32, LANES) * NBINS
        ones16 = jnp.ones((LANES,), jnp.int32)

        @pl.loop(0, PER_W, step=CH)
        def _chunk(c):
            pltpu.sync_copy(x_hbm.at[pl.ds(base + c, CH)], buf_v)

            @pl.loop(0, CH, step=LANES)
            def _vec(i):
                v = buf_v[pl.ds(i, LANES)]
                u = lax.bitcast_convert_type(v, jnp.int32)
                key = _key_i32(u)
                bin_ = ((key >> shift) + bias) & jnp.int32(NBINS - 1)
                idx = bin_ + lane_base
                if masked:
                    keep = (key >> mshift) == mt
                    plsc.addupdate_scatter(hist_v, [idx], ones16, mask=keep)
                else:
                    plsc.addupdate_scatter(hist_v, [idx], ones16)

        pltpu.sync_copy(hist_v, out_hbm.at[wid])

    scratch = [
        pltpu.VMEM((HIST,), jnp.int32),
        pltpu.VMEM((CH,), jnp.float32),
        pltpu.VMEM((LANES,), jnp.int32),
        pltpu.SemaphoreType.DMA,
    ]
    return pl.kernel(
        body,
        out_type=jax.ShapeDtypeStruct((NW, HIST), jnp.int32),
        mesh=_mesh,
        scratch_types=scratch,
        compiler_params=pltpu.CompilerParams(needs_layout_passes=False),
    )


_sc_pass1 = _make_sc_hist(shift=20, bias=2048, mshift=0, masked=False)
_sc_pass2 = _make_sc_hist(shift=8, bias=0, mshift=20, masked=True)
_sc_pass3 = _make_sc_hist(shift=0, bias=0, mshift=8, masked=True)


def _make_tc_scan(stage):
    def body(hist_ref, *rest):
        if stage == 1:
            (out_ref,) = rest
            need = jnp.int32(TOTAL_K)
        else:
            prev_ref, out_ref = rest
            need = prev_ref[0, 1]

        cnt = jnp.sum(hist_ref[...], axis=0, keepdims=True)
        iota_b = lax.broadcasted_iota(jnp.int32, (1, NBINS), 1)

        def suffix(bb):
            return jnp.sum(jnp.where(iota_b >= bb, cnt, 0))

        def step(_, carry):
            lo, hi = carry
            mid = (lo + hi) // 2
            go = suffix(mid) >= need
            return (jnp.where(go, mid, lo), jnp.where(go, hi, mid))

        lo, _hi = lax.fori_loop(0, 12, step, (jnp.int32(0), jnp.int32(NBINS)))
        bstar = lo
        g = suffix(bstar + 1)
        r = need - g
        cnt_b = suffix(bstar) - g

        if stage == 1:
            v0 = bstar - jnp.int32(2048)
            v1 = r
            v2 = jnp.int32(0)
        elif stage == 2:
            prev_t = prev_ref[0, 0]
            v0 = prev_t * jnp.int32(NBINS) + bstar
            v1 = r
            v2 = jnp.int32(0)
        else:
            prev_t = prev_ref[0, 0]
            v0 = prev_t * jnp.int32(256) + (bstar & jnp.int32(255))
            v1 = r
            v2 = (r < cnt_b).astype(jnp.int32)

        iota_o = lax.broadcasted_iota(jnp.int32, (1, 128), 1)
        out_ref[...] = (
            jnp.where(iota_o == 0, v0, 0)
            + jnp.where(iota_o == 1, v1, 0)
            + jnp.where(iota_o == 2, v2, 0)
        )

    n_in = 1 if stage == 1 else 2
    in_specs = [pl.BlockSpec((NW * 16, NBINS), lambda: (0, 0))]
    if n_in == 2:
        in_specs.append(
            pl.BlockSpec((1, 128), lambda: (0, 0), memory_space=pltpu.SMEM)
        )
    return pl.pallas_call(
        body,
        grid=(),
        in_specs=in_specs,
        out_specs=pl.BlockSpec((1, 128), lambda: (0, 0)),
        out_shape=jax.ShapeDtypeStruct((1, 128), jnp.int32),
    )


_tc_scan1 = _make_tc_scan(1)
_tc_scan2 = _make_tc_scan(2)
_tc_scan3 = _make_tc_scan(3)

_ROWS_PER_BLK = 8
_NBLK = B // _ROWS_PER_BLK


def _cumsum_lanes(x):
    acc = x
    lane = lax.broadcasted_iota(jnp.int32, x.shape, 1)
    s = 1
    while s < x.shape[1]:
        acc = acc + jnp.where(lane >= s, pltpu.roll(acc, s, 1), 0)
        s *= 2
    return acc


def _cumsum_sublanes_excl(x):
    acc = x
    sub = lax.broadcasted_iota(jnp.int32, x.shape, 0)
    s = 1
    while s < x.shape[0]:
        acc = acc + jnp.where(sub >= s, pltpu.roll(acc, s, 0), 0)
        s *= 2
    return acc - x


def _tc_out_body(x_ref, p_ref, o_ref, carry_ref):
    pid = pl.program_id(0)
    t_key = p_ref[0, 0]
    t_cnt = p_ref[0, 1]
    tie = p_ref[0, 2]

    @pl.when(pid == 0)
    def _init():
        carry_ref[0] = 0

    xv = x_ref[...]
    u = lax.bitcast_convert_type(xv, jnp.int32)
    key = _key_i32(u)

    @pl.when(tie == 0)
    def _fast():
        o_ref[...] = jnp.where(key >= t_key, xv, 0.0)

    @pl.when(tie != 0)
    def _tiebreak():
        eq = (key == t_key).astype(jnp.int32)
        cs = _cumsum_lanes(eq)
        row_tot = cs[:, F - 1 : F]
        row_off = _cumsum_sublanes_excl(row_tot)
        rank = cs - eq + row_off + carry_ref[0]
        keep = (key > t_key) | ((eq == 1) & (rank < t_cnt))
        o_ref[...] = jnp.where(keep, xv, 0.0)
        carry_ref[0] = carry_ref[0] + jnp.sum(eq)


_tc_out = pl.pallas_call(
    _tc_out_body,
    grid=(_NBLK,),
    in_specs=[
        pl.BlockSpec((_ROWS_PER_BLK, F), lambda i: (i, 0)),
        pl.BlockSpec((1, 128), lambda i: (0, 0), memory_space=pltpu.SMEM),
    ],
    out_specs=pl.BlockSpec((_ROWS_PER_BLK, F), lambda i: (i, 0)),
    out_shape=jax.ShapeDtypeStruct((B, F), jnp.float32),
    scratch_shapes=[pltpu.SMEM((1,), jnp.int32)],
)


def kernel(x):
    xf = x.reshape(N)

    h1 = _sc_pass1(xf)
    p1 = _tc_scan1(h1.reshape(NW * 16, NBINS))

    mt1 = jnp.full((LANES,), p1[0, 0], jnp.int32)
    h2 = _sc_pass2(xf, mt1)
    p2 = _tc_scan2(h2.reshape(NW * 16, NBINS), p1)

    mt2 = jnp.full((LANES,), p2[0, 0], jnp.int32)
    h3 = _sc_pass3(xf, mt2)
    p3 = _tc_scan3(h3.reshape(NW * 16, NBINS), p2)

    return _tc_out(x, p3)
